# baseline (speedup 1.0000x reference)
import functools

import jax
import jax.numpy as jnp
from jax import lax
from jax.experimental import pallas as pl
from jax.experimental.pallas import tpu as pltpu
from jax.experimental.pallas import tpu_sc as plsc

N = 10000
D = 128
NPAD = 10240
E = 320000
EPAD = 327680
NC1, NS = 2, 16
ROWS_PER_TILE = NPAD // NS
CH1 = 128
NJ1 = EPAD // (NC1 * NS) // CH1
NJ2 = EPAD // NS // CH1

_mesh1 = plsc.VectorSubcoreMesh(
    core_axis_name="c", subcore_axis_name="s", num_cores=2, num_subcores=16)
_mesh2 = plsc.VectorSubcoreMesh(
    core_axis_name="c", subcore_axis_name="s", num_cores=1, num_subcores=16)


DH = D // 2
NBUF = 4
NJP1 = EPAD // NS // CH1


def _p1_body(src_h, dst_h, feat_h, zeros_h, agg_h, degp_h,
             src_v, dst_v, rows_v, deg_v, acc_s, sem_g, sem_s):
    c = lax.axis_index("c")
    s = lax.axis_index("s")
    wid = s
    z16 = jnp.zeros((16,), jnp.float32)
    o16 = jnp.ones((16,), jnp.float32)

    def _zdeg(i, _):
        deg_v[pl.ds(i * 16, 16)] = z16
        return 0
    lax.fori_loop(0, NPAD // 16, _zdeg, 0)

    pltpu.sync_copy(src_h.at[wid], src_v)
    pltpu.sync_copy(dst_h.at[wid], dst_v)

    base = pl.multiple_of(s * ROWS_PER_TILE, 128)
    pltpu.sync_copy(zeros_h.at[pl.ds(base, ROWS_PER_TILE)],
                    acc_s.at[pl.ds(base, ROWS_PER_TILE)])
    plsc.subcore_barrier()

    rows = tuple(rows_v.at[b] for b in range(NBUF))
    gsem = tuple(sem_g.at[b] for b in range(NBUF))
    ssem = tuple(sem_s.at[b] for b in range(NBUF))

    def _gather(j, b):
        return pltpu.async_copy(feat_h.at[src_v.at[j]], rows[b], gsem[b])

    def _scatter(j, b):
        return pltpu.async_copy(rows[b], acc_s.at[dst_v.at[j]], ssem[b],
                                add=True)

    def _hist(j):
        for k in range(8):
            d16 = dst_v[j, pl.ds(k * 16, 16)]
            plsc.addupdate_scatter(deg_v, [d16], o16)

    def _wait_scatter(j, b):
        pltpu.make_async_copy(rows[b], acc_s.at[dst_v.at[j]],
                              ssem[b]).wait()

    def _wait_gather(j, b):
        pltpu.make_async_copy(feat_h.at[src_v.at[j]], rows[b],
                              gsem[b]).wait()

    for b in range(NBUF):
        _gather(b, b)
        _hist(b)
    for b in range(NBUF):
        _wait_gather(b, b)
        _scatter(b, b)

    def _grp(jj, _):
        j0 = jj * NBUF
        for b in range(NBUF):
            _wait_scatter(j0 + b - NBUF, b)
            _gather(j0 + b, b)
            _hist(j0 + b)
        for b in range(NBUF):
            _wait_gather(j0 + b, b)
            _scatter(j0 + b, b)
        return 0
    lax.fori_loop(1, NJP1 // NBUF, _grp, 0)
    for b in range(NBUF):
        _wait_scatter(NJP1 - NBUF + b, b)

    plsc.subcore_barrier()
    pltpu.sync_copy(acc_s.at[pl.ds(base, ROWS_PER_TILE)],
                    agg_h.at[c, pl.ds(base, ROWS_PER_TILE)])
    pltpu.sync_copy(deg_v, degp_h.at[wid])


_pass1 = pl.kernel(
    _p1_body,
    out_type=[
        jax.ShapeDtypeStruct((1, NPAD, D), jnp.bfloat16),
        jax.ShapeDtypeStruct((NS, NPAD), jnp.float32),
    ],
    mesh=_mesh2,
    scratch_types=[
        pltpu.VMEM((NJP1, CH1), jnp.int32),
        pltpu.VMEM((NJP1, CH1), jnp.int32),
        pltpu.VMEM((NBUF, CH1, D), jnp.bfloat16),
        pltpu.VMEM((NPAD,), jnp.float32),
        pltpu.VMEM_SHARED((NPAD, D), jnp.bfloat16),
        pltpu.SemaphoreType.DMA((NBUF,)),
        pltpu.SemaphoreType.DMA((NBUF,)),
    ],
    compiler_params=pltpu.CompilerParams(
        needs_layout_passes=False, use_tc_tiling_on_sc=False),
)


def _dense_body(feat, aggp, degp, w1s, w1n, w2s, w2n, b1s, b1n, b2s, b2n,
                pre_o, pn_o, deg_o):
    deg = jnp.sum(degp[...], axis=0)
    degc = jnp.maximum(deg, 1.0)
    agg = aggp[0].astype(jnp.float32)
    mean = agg * (1.0 / degc)[:, None]
    hp = jax.lax.Precision.HIGHEST
    x = (jnp.dot(feat[...], w1s[...], precision=hp)
         + jnp.dot(mean, w1n[...], precision=hp)
         + b1s[...] + b1n[...])
    x = jnp.maximum(x, 0.0)
    pre_o[...] = (jnp.sum(x * w2s[...], axis=1, keepdims=True)
                  + b2s[...] + b2n[...])
    pn_o[...] = jnp.sum(x * w2n[...], axis=1, keepdims=True)
    deg_o[...] = deg[:, None]


_R = 2048


def _dense(featp, agg, degp, W1_self, W1_neigh, w2s, w2n, b1s, b1n, b2s, b2n):
    grid = (NPAD // _R,)
    full = lambda *shape: pl.BlockSpec(shape, lambda i: (0,) * len(shape))
    return pl.pallas_call(
        _dense_body,
        grid=grid,
        in_specs=[
            pl.BlockSpec((_R, D), lambda i: (i, 0)),
            pl.BlockSpec((1, _R, D), lambda i: (0, i, 0)),
            pl.BlockSpec((NS, _R), lambda i: (0, i)),
            full(D, D), full(D, D), full(1, D), full(1, D),
            full(1, D), full(1, D), full(1, 1), full(1, 1),
        ],
        out_specs=[
            pl.BlockSpec((_R, 1), lambda i: (i, 0)),
            pl.BlockSpec((_R, 1), lambda i: (i, 0)),
            pl.BlockSpec((_R, 1), lambda i: (i, 0)),
        ],
        out_shape=[
            jax.ShapeDtypeStruct((NPAD, 1), jnp.float32),
            jax.ShapeDtypeStruct((NPAD, 1), jnp.float32),
            jax.ShapeDtypeStruct((NPAD, 1), jnp.float32),
        ],
    )(featp, agg, degp, W1_self, W1_neigh, w2s, w2n, b1s, b1n, b2s, b2n)


def _p2_body(src_h, dst_h, p_h, out_h,
             p_v, acc_v, src_v, dst_v, part_v, sum_v, stage_s):
    c = lax.axis_index("c")
    t = lax.axis_index("s")
    wid = t * NC1 + c
    z16 = jnp.zeros((16,), jnp.float32)

    pltpu.sync_copy(p_h, p_v)

    def _zacc(i, _):
        acc_v[pl.ds(i * 16, 16)] = z16
        return 0
    lax.fori_loop(0, NPAD // 16, _zacc, 0)

    pltpu.sync_copy(src_h.at[wid], src_v)
    pltpu.sync_copy(dst_h.at[wid], dst_v)

    def _step(j, _):
        for k in range(8):
            s16 = src_v[j, pl.ds(k * 16, 16)]
            d16 = dst_v[j, pl.ds(k * 16, 16)]
            vals = plsc.load_gather(p_v, [s16])
            plsc.addupdate_scatter(acc_v, [d16], vals)
        return 0
    lax.fori_loop(0, NJ1, _step, 0)

    pltpu.sync_copy(acc_v, stage_s.at[t])
    plsc.subcore_barrier()

    base = pl.multiple_of(t * ROWS_PER_TILE, 128)
    for r in range(NS):
        pltpu.sync_copy(stage_s.at[r, pl.ds(base, ROWS_PER_TILE)], part_v.at[r])

    def _red(m, _):
        a = part_v[0, pl.ds(m * 16, 16)]
        for r in range(1, NS):
            a = a + part_v[r, pl.ds(m * 16, 16)]
        sum_v[pl.ds(m * 16, 16)] = a
        return 0
    lax.fori_loop(0, ROWS_PER_TILE // 16, _red, 0)

    pltpu.sync_copy(sum_v, out_h.at[c, pl.ds(base, ROWS_PER_TILE)])


_pass2 = pl.kernel(
    _p2_body,
    out_type=jax.ShapeDtypeStruct((NC1, NPAD), jnp.float32),
    mesh=_mesh1,
    scratch_types=[
        pltpu.VMEM((NPAD,), jnp.float32),
        pltpu.VMEM((NPAD,), jnp.float32),
        pltpu.VMEM((NJ1, CH1), jnp.int32),
        pltpu.VMEM((NJ1, CH1), jnp.int32),
        pltpu.VMEM((NS, ROWS_PER_TILE), jnp.float32),
        pltpu.VMEM((ROWS_PER_TILE,), jnp.float32),
        pltpu.VMEM_SHARED((NS, NPAD), jnp.float32),
    ],
    compiler_params=pltpu.CompilerParams(needs_layout_passes=False),
)


def _fin_body(part, pre, deg, o_ref):
    ps = part[0] + part[1]
    z = pre[...] + ps / jnp.maximum(deg[...], 1.0)
    o_ref[...] = 1.0 / (1.0 + jnp.exp(-z))


_finalize = pl.pallas_call(
    _fin_body,
    out_shape=jax.ShapeDtypeStruct((NPAD // 128, 128), jnp.float32),
)


def kernel(features, edge_index, edge_types,
           W1_self, b1_self, W1_neigh, b1_neigh,
           W2_self, b2_self, W2_neigh, b2_neigh):
    src = edge_index[0].astype(jnp.int32)
    dst = edge_index[1].astype(jnp.int32)
    srcp = jnp.concatenate(
        [src, jnp.zeros((EPAD - E,), jnp.int32)])
    dstp = jnp.concatenate(
        [dst, jnp.full((EPAD - E,), NPAD - 1, jnp.int32)])
    src1 = srcp.reshape(NC1 * NS, NJ1, CH1)
    dst1 = dstp.reshape(NC1 * NS, NJ1, CH1)
    src1c = srcp.reshape(NS, NJP1, CH1)
    dst1c = dstp.reshape(NS, NJP1, CH1)
    featp = jnp.pad(features, ((0, NPAD - N), (0, 0)))
    featbf = featp.astype(jnp.bfloat16)
    zeros = jnp.zeros((NPAD, D), jnp.bfloat16)

    agg, degp = _pass1(src1c, dst1c, featbf, zeros)
    pre, pn, deg = _dense(
        featp, agg, degp, W1_self, W1_neigh,
        W2_self.reshape(1, D), W2_neigh.reshape(1, D),
        b1_self.reshape(1, D), b1_neigh.reshape(1, D),
        b2_self.reshape(1, 1), b2_neigh.reshape(1, 1))
    part = _pass2(src1, dst1, pn.reshape(NPAD))
    out = _finalize(part.reshape(NC1, NPAD // 128, 128),
                    pre.reshape(NPAD // 128, 128),
                    deg.reshape(NPAD // 128, 128))
    return out.reshape(NPAD)[:N]


# speedup vs baseline: 1.6384x; 1.6384x over previous
import functools

import jax
import jax.numpy as jnp
from jax import lax
from jax.experimental import pallas as pl
from jax.experimental.pallas import tpu as pltpu
from jax.experimental.pallas import tpu_sc as plsc

N = 10000
D = 128
NPAD = 10240
E = 320000
EPAD = 327680
NC1, NS = 2, 16
ROWS_PER_TILE = NPAD // NS
CH1 = 128
NJ1 = EPAD // (NC1 * NS) // CH1
NJ2 = EPAD // NS // CH1

_mesh1 = plsc.VectorSubcoreMesh(
    core_axis_name="c", subcore_axis_name="s", num_cores=2, num_subcores=16)
_mesh2 = plsc.VectorSubcoreMesh(
    core_axis_name="c", subcore_axis_name="s", num_cores=1, num_subcores=16)


DH = D // 2
NBUF = 4
NJP1 = EPAD // NS // CH1


def _p1_body(src_h, dst_h, feat_h, zeros_h, agg_h, degp_h,
             src_v, dst_v, rows_v, deg_v, tab_s, acc_s, sem_g, sem_s):
    c = lax.axis_index("c")
    s = lax.axis_index("s")
    wid = s * NC1 + c
    z16 = jnp.zeros((16,), jnp.float32)
    o16 = jnp.ones((16,), jnp.float32)

    def _zdeg(i, _):
        deg_v[pl.ds(i * 16, 16)] = z16
        return 0
    lax.fori_loop(0, NPAD // 16, _zdeg, 0)

    pltpu.sync_copy(src_h.at[wid], src_v)
    pltpu.sync_copy(dst_h.at[wid], dst_v)

    base = pl.multiple_of(s * ROWS_PER_TILE, 128)

    rows = tuple(rows_v.at[b] for b in range(NBUF))
    gsem = tuple(sem_g.at[b] for b in range(NBUF))
    ssem = tuple(sem_s.at[b] for b in range(NBUF))

    for h in range(2):
        pltpu.sync_copy(feat_h.at[h, pl.ds(base, ROWS_PER_TILE)],
                        tab_s.at[pl.ds(base, ROWS_PER_TILE)])
        pltpu.sync_copy(zeros_h.at[pl.ds(base, ROWS_PER_TILE)],
                        acc_s.at[pl.ds(base, ROWS_PER_TILE)])
        plsc.subcore_barrier()

        def _gather(j, b):
            return pltpu.async_copy(tab_s.at[src_v.at[j]], rows[b], gsem[b])

        def _scatter(j, b):
            return pltpu.async_copy(rows[b], acc_s.at[dst_v.at[j]], ssem[b],
                                    add=True)

        def _hist(j):
            for k in range(8):
                d16 = dst_v[j, pl.ds(k * 16, 16)]
                plsc.addupdate_scatter(deg_v, [d16], o16)

        def _wait_scatter(j, b):
            pltpu.make_async_copy(rows[b], acc_s.at[dst_v.at[j]],
                                  ssem[b]).wait()

        def _wait_gather(j, b):
            pltpu.make_async_copy(tab_s.at[src_v.at[j]], rows[b],
                                  gsem[b]).wait()

        for b in range(NBUF):
            _gather(b, b)
            if h == 0:
                _hist(b)
        for b in range(NBUF):
            _wait_gather(b, b)
            _scatter(b, b)

        def _grp(jj, _):
            j0 = jj * NBUF
            for b in range(NBUF):
                _wait_scatter(j0 + b - NBUF, b)
                _gather(j0 + b, b)
            if h == 0:
                for b in range(NBUF):
                    _hist(j0 + b)
            for b in range(NBUF):
                _wait_gather(j0 + b, b)
                _scatter(j0 + b, b)
            return 0
        lax.fori_loop(1, NJ1 // NBUF, _grp, 0)
        for b in range(NBUF):
            _wait_scatter(NJ1 - NBUF + b, b)

        plsc.subcore_barrier()
        pltpu.sync_copy(acc_s.at[pl.ds(base, ROWS_PER_TILE)],
                        agg_h.at[c, h, pl.ds(base, ROWS_PER_TILE)])
        plsc.subcore_barrier()
    pltpu.sync_copy(deg_v, degp_h.at[wid])


_pass1 = pl.kernel(
    _p1_body,
    out_type=[
        jax.ShapeDtypeStruct((NC1, 2, NPAD, DH), jnp.bfloat16),
        jax.ShapeDtypeStruct((NC1 * NS, NPAD), jnp.float32),
    ],
    mesh=_mesh1,
    scratch_types=[
        pltpu.VMEM((NJ1, CH1), jnp.int32),
        pltpu.VMEM((NJ1, CH1), jnp.int32),
        pltpu.VMEM((NBUF, CH1, DH), jnp.bfloat16),
        pltpu.VMEM((NPAD,), jnp.float32),
        pltpu.VMEM_SHARED((NPAD, DH), jnp.bfloat16),
        pltpu.VMEM_SHARED((NPAD, DH), jnp.bfloat16),
        pltpu.SemaphoreType.DMA((NBUF,)),
        pltpu.SemaphoreType.DMA((NBUF,)),
    ],
    compiler_params=pltpu.CompilerParams(
        needs_layout_passes=False, use_tc_tiling_on_sc=False),
)


def _dense_body(feat, aggp, degp, w1s, w1n, w2s, w2n, b1s, b1n, b2s, b2n,
                pre_o, pn_o, deg_o):
    deg = jnp.sum(degp[...], axis=0)
    degc = jnp.maximum(deg, 1.0)
    agg = jnp.concatenate(
        [aggp[0, 0].astype(jnp.float32) + aggp[1, 0].astype(jnp.float32),
         aggp[0, 1].astype(jnp.float32) + aggp[1, 1].astype(jnp.float32)],
        axis=1)
    mean = agg * (1.0 / degc)[:, None]
    hp = jax.lax.Precision.HIGHEST
    x = (jnp.dot(feat[...], w1s[...], precision=hp)
         + jnp.dot(mean, w1n[...], precision=hp)
         + b1s[...] + b1n[...])
    x = jnp.maximum(x, 0.0)
    pre_o[...] = (jnp.sum(x * w2s[...], axis=1, keepdims=True)
                  + b2s[...] + b2n[...])
    pn_o[...] = jnp.sum(x * w2n[...], axis=1, keepdims=True)
    deg_o[...] = deg[:, None]


_R = 2048


def _dense(featp, agg, degp, W1_self, W1_neigh, w2s, w2n, b1s, b1n, b2s, b2n):
    grid = (NPAD // _R,)
    full = lambda *shape: pl.BlockSpec(shape, lambda i: (0,) * len(shape))
    return pl.pallas_call(
        _dense_body,
        grid=grid,
        in_specs=[
            pl.BlockSpec((_R, D), lambda i: (i, 0)),
            pl.BlockSpec((NC1, 2, _R, DH), lambda i: (0, 0, i, 0)),
            pl.BlockSpec((NC1 * NS, _R), lambda i: (0, i)),
            full(D, D), full(D, D), full(1, D), full(1, D),
            full(1, D), full(1, D), full(1, 1), full(1, 1),
        ],
        out_specs=[
            pl.BlockSpec((_R, 1), lambda i: (i, 0)),
            pl.BlockSpec((_R, 1), lambda i: (i, 0)),
            pl.BlockSpec((_R, 1), lambda i: (i, 0)),
        ],
        out_shape=[
            jax.ShapeDtypeStruct((NPAD, 1), jnp.float32),
            jax.ShapeDtypeStruct((NPAD, 1), jnp.float32),
            jax.ShapeDtypeStruct((NPAD, 1), jnp.float32),
        ],
    )(featp, agg, degp, W1_self, W1_neigh, w2s, w2n, b1s, b1n, b2s, b2n)


def _p2_body(src_h, dst_h, p_h, out_h,
             p_v, acc_v, src_v, dst_v, part_v, sum_v, stage_s):
    c = lax.axis_index("c")
    t = lax.axis_index("s")
    wid = t * NC1 + c
    z16 = jnp.zeros((16,), jnp.float32)

    pltpu.sync_copy(p_h, p_v)

    def _zacc(i, _):
        acc_v[pl.ds(i * 16, 16)] = z16
        return 0
    lax.fori_loop(0, NPAD // 16, _zacc, 0)

    pltpu.sync_copy(src_h.at[wid], src_v)
    pltpu.sync_copy(dst_h.at[wid], dst_v)

    def _step(j, _):
        for k in range(8):
            s16 = src_v[j, pl.ds(k * 16, 16)]
            d16 = dst_v[j, pl.ds(k * 16, 16)]
            vals = plsc.load_gather(p_v, [s16])
            plsc.addupdate_scatter(acc_v, [d16], vals)
        return 0
    lax.fori_loop(0, NJ1, _step, 0)

    pltpu.sync_copy(acc_v, stage_s.at[t])
    plsc.subcore_barrier()

    base = pl.multiple_of(t * ROWS_PER_TILE, 128)
    for r in range(NS):
        pltpu.sync_copy(stage_s.at[r, pl.ds(base, ROWS_PER_TILE)], part_v.at[r])

    def _red(m, _):
        a = part_v[0, pl.ds(m * 16, 16)]
        for r in range(1, NS):
            a = a + part_v[r, pl.ds(m * 16, 16)]
        sum_v[pl.ds(m * 16, 16)] = a
        return 0
    lax.fori_loop(0, ROWS_PER_TILE // 16, _red, 0)

    pltpu.sync_copy(sum_v, out_h.at[c, pl.ds(base, ROWS_PER_TILE)])


_pass2 = pl.kernel(
    _p2_body,
    out_type=jax.ShapeDtypeStruct((NC1, NPAD), jnp.float32),
    mesh=_mesh1,
    scratch_types=[
        pltpu.VMEM((NPAD,), jnp.float32),
        pltpu.VMEM((NPAD,), jnp.float32),
        pltpu.VMEM((NJ1, CH1), jnp.int32),
        pltpu.VMEM((NJ1, CH1), jnp.int32),
        pltpu.VMEM((NS, ROWS_PER_TILE), jnp.float32),
        pltpu.VMEM((ROWS_PER_TILE,), jnp.float32),
        pltpu.VMEM_SHARED((NS, NPAD), jnp.float32),
    ],
    compiler_params=pltpu.CompilerParams(needs_layout_passes=False),
)


def _fin_body(part, pre, deg, o_ref):
    ps = part[0] + part[1]
    z = pre[...] + ps / jnp.maximum(deg[...], 1.0)
    o_ref[...] = 1.0 / (1.0 + jnp.exp(-z))


_finalize = pl.pallas_call(
    _fin_body,
    out_shape=jax.ShapeDtypeStruct((NPAD // 128, 128), jnp.float32),
)


def kernel(features, edge_index, edge_types,
           W1_self, b1_self, W1_neigh, b1_neigh,
           W2_self, b2_self, W2_neigh, b2_neigh):
    src = edge_index[0].astype(jnp.int32)
    dst = edge_index[1].astype(jnp.int32)
    srcp = jnp.concatenate(
        [src, jnp.zeros((EPAD - E,), jnp.int32)])
    dstp = jnp.concatenate(
        [dst, jnp.full((EPAD - E,), NPAD - 1, jnp.int32)])
    src1 = srcp.reshape(NC1 * NS, NJ1, CH1)
    dst1 = dstp.reshape(NC1 * NS, NJ1, CH1)
    src1c = srcp.reshape(NS, NJP1, CH1)
    dst1c = dstp.reshape(NS, NJP1, CH1)
    featp = jnp.pad(features, ((0, NPAD - N), (0, 0)))
    featbf = featp.astype(jnp.bfloat16)
    feat2 = featbf.reshape(NPAD, 2, DH).transpose(1, 0, 2)
    zeros = jnp.zeros((NPAD, DH), jnp.bfloat16)

    agg, degp = _pass1(src1, dst1, feat2, zeros)
    pre, pn, deg = _dense(
        featp, agg, degp, W1_self, W1_neigh,
        W2_self.reshape(1, D), W2_neigh.reshape(1, D),
        b1_self.reshape(1, D), b1_neigh.reshape(1, D),
        b2_self.reshape(1, 1), b2_neigh.reshape(1, 1))
    part = _pass2(src1, dst1, pn.reshape(NPAD))
    out = _finalize(part.reshape(NC1, NPAD // 128, 128),
                    pre.reshape(NPAD // 128, 128),
                    deg.reshape(NPAD // 128, 128))
    return out.reshape(NPAD)[:N]


# speedup vs baseline: 1.6876x; 1.0300x over previous
import functools

import jax
import jax.numpy as jnp
from jax import lax
from jax.experimental import pallas as pl
from jax.experimental.pallas import tpu as pltpu
from jax.experimental.pallas import tpu_sc as plsc

N = 10000
D = 128
NPAD = 10240
E = 320000
EPAD = 327680
NC1, NS = 2, 16
ROWS_PER_TILE = NPAD // NS
CH1 = 128
NJ1 = EPAD // (NC1 * NS) // CH1
NJ2 = EPAD // NS // CH1

_mesh1 = plsc.VectorSubcoreMesh(
    core_axis_name="c", subcore_axis_name="s", num_cores=2, num_subcores=16)
_mesh2 = plsc.VectorSubcoreMesh(
    core_axis_name="c", subcore_axis_name="s", num_cores=1, num_subcores=16)


DH = D // 2
NBUF = 8
NJP1 = EPAD // NS // CH1


def _p1_body(src_h, dst_h, feat_h, zeros_h, agg_h, degp_h,
             src_v, dst_v, rows_v, deg_v, tab_s, acc_s, sem_g, sem_s):
    c = lax.axis_index("c")
    s = lax.axis_index("s")
    wid = s * NC1 + c
    z16 = jnp.zeros((16,), jnp.float32)
    o16 = jnp.ones((16,), jnp.float32)

    def _zdeg(i, _):
        deg_v[pl.ds(i * 16, 16)] = z16
        return 0
    lax.fori_loop(0, NPAD // 16, _zdeg, 0)

    pltpu.sync_copy(src_h.at[wid], src_v)
    pltpu.sync_copy(dst_h.at[wid], dst_v)

    base = pl.multiple_of(s * ROWS_PER_TILE, 128)

    rows = tuple(rows_v.at[b] for b in range(NBUF))
    gsem = tuple(sem_g.at[b] for b in range(NBUF))
    ssem = tuple(sem_s.at[b] for b in range(NBUF))

    for h in range(2):
        pltpu.sync_copy(feat_h.at[h, pl.ds(base, ROWS_PER_TILE)],
                        tab_s.at[pl.ds(base, ROWS_PER_TILE)])
        pltpu.sync_copy(zeros_h.at[pl.ds(base, ROWS_PER_TILE)],
                        acc_s.at[pl.ds(base, ROWS_PER_TILE)])
        plsc.subcore_barrier()

        def _gather(j, b):
            return pltpu.async_copy(tab_s.at[src_v.at[j]], rows[b], gsem[b])

        def _scatter(j, b):
            return pltpu.async_copy(rows[b], acc_s.at[dst_v.at[j]], ssem[b],
                                    add=True)

        def _hist(j):
            for k in range(8):
                d16 = dst_v[j, pl.ds(k * 16, 16)]
                plsc.addupdate_scatter(deg_v, [d16], o16)

        def _wait_scatter(j, b):
            pltpu.make_async_copy(rows[b], acc_s.at[dst_v.at[j]],
                                  ssem[b]).wait()

        def _wait_gather(j, b):
            pltpu.make_async_copy(tab_s.at[src_v.at[j]], rows[b],
                                  gsem[b]).wait()

        for b in range(NBUF):
            _gather(b, b)
            if h == 0:
                _hist(b)
        for b in range(NBUF):
            _wait_gather(b, b)
            _scatter(b, b)

        def _grp(jj, _):
            j0 = jj * NBUF
            for b in range(NBUF):
                _wait_scatter(j0 + b - NBUF, b)
                _gather(j0 + b, b)
            if h == 0:
                for b in range(NBUF):
                    _hist(j0 + b)
            for b in range(NBUF):
                _wait_gather(j0 + b, b)
                _scatter(j0 + b, b)
            return 0
        lax.fori_loop(1, NJ1 // NBUF, _grp, 0)
        for b in range(NBUF):
            _wait_scatter(NJ1 - NBUF + b, b)

        plsc.subcore_barrier()
        pltpu.sync_copy(acc_s.at[pl.ds(base, ROWS_PER_TILE)],
                        agg_h.at[c, h, pl.ds(base, ROWS_PER_TILE)])
        plsc.subcore_barrier()
    pltpu.sync_copy(deg_v, degp_h.at[wid])


_pass1 = pl.kernel(
    _p1_body,
    out_type=[
        jax.ShapeDtypeStruct((NC1, 2, NPAD, DH), jnp.bfloat16),
        jax.ShapeDtypeStruct((NC1 * NS, NPAD), jnp.float32),
    ],
    mesh=_mesh1,
    scratch_types=[
        pltpu.VMEM((NJ1, CH1), jnp.int32),
        pltpu.VMEM((NJ1, CH1), jnp.int32),
        pltpu.VMEM((NBUF, CH1, DH), jnp.bfloat16),
        pltpu.VMEM((NPAD,), jnp.float32),
        pltpu.VMEM_SHARED((NPAD, DH), jnp.bfloat16),
        pltpu.VMEM_SHARED((NPAD, DH), jnp.bfloat16),
        pltpu.SemaphoreType.DMA((NBUF,)),
        pltpu.SemaphoreType.DMA((NBUF,)),
    ],
    compiler_params=pltpu.CompilerParams(
        needs_layout_passes=False, use_tc_tiling_on_sc=False),
)


def _dense_body(feat, aggp, degp, w1s, w1n, w2s, w2n, b1s, b1n, b2s, b2n,
                pre_o, pn_o, deg_o):
    deg = jnp.sum(degp[...], axis=0)
    degc = jnp.maximum(deg, 1.0)
    agg = jnp.concatenate(
        [aggp[0, 0].astype(jnp.float32) + aggp[1, 0].astype(jnp.float32),
         aggp[0, 1].astype(jnp.float32) + aggp[1, 1].astype(jnp.float32)],
        axis=1)
    mean = agg * (1.0 / degc)[:, None]
    hp = jax.lax.Precision.HIGHEST
    x = (jnp.dot(feat[...], w1s[...], precision=hp)
         + jnp.dot(mean, w1n[...], precision=hp)
         + b1s[...] + b1n[...])
    x = jnp.maximum(x, 0.0)
    pre_o[...] = (jnp.sum(x * w2s[...], axis=1, keepdims=True)
                  + b2s[...] + b2n[...])
    pn_o[...] = jnp.sum(x * w2n[...], axis=1, keepdims=True)
    deg_o[...] = deg[:, None]


_R = 2048


def _dense(featp, agg, degp, W1_self, W1_neigh, w2s, w2n, b1s, b1n, b2s, b2n):
    grid = (NPAD // _R,)
    full = lambda *shape: pl.BlockSpec(shape, lambda i: (0,) * len(shape))
    return pl.pallas_call(
        _dense_body,
        grid=grid,
        in_specs=[
            pl.BlockSpec((_R, D), lambda i: (i, 0)),
            pl.BlockSpec((NC1, 2, _R, DH), lambda i: (0, 0, i, 0)),
            pl.BlockSpec((NC1 * NS, _R), lambda i: (0, i)),
            full(D, D), full(D, D), full(1, D), full(1, D),
            full(1, D), full(1, D), full(1, 1), full(1, 1),
        ],
        out_specs=[
            pl.BlockSpec((_R, 1), lambda i: (i, 0)),
            pl.BlockSpec((_R, 1), lambda i: (i, 0)),
            pl.BlockSpec((_R, 1), lambda i: (i, 0)),
        ],
        out_shape=[
            jax.ShapeDtypeStruct((NPAD, 1), jnp.float32),
            jax.ShapeDtypeStruct((NPAD, 1), jnp.float32),
            jax.ShapeDtypeStruct((NPAD, 1), jnp.float32),
        ],
    )(featp, agg, degp, W1_self, W1_neigh, w2s, w2n, b1s, b1n, b2s, b2n)


def _p2_body(src_h, dst_h, p_h, out_h,
             p_v, acc_v, src_v, dst_v, part_v, sum_v, stage_s):
    c = lax.axis_index("c")
    t = lax.axis_index("s")
    wid = t * NC1 + c
    z16 = jnp.zeros((16,), jnp.float32)

    pltpu.sync_copy(p_h, p_v)

    def _zacc(i, _):
        acc_v[pl.ds(i * 16, 16)] = z16
        return 0
    lax.fori_loop(0, NPAD // 16, _zacc, 0)

    pltpu.sync_copy(src_h.at[wid], src_v)
    pltpu.sync_copy(dst_h.at[wid], dst_v)

    def _step(j, _):
        for k in range(8):
            s16 = src_v[j, pl.ds(k * 16, 16)]
            d16 = dst_v[j, pl.ds(k * 16, 16)]
            vals = plsc.load_gather(p_v, [s16])
            plsc.addupdate_scatter(acc_v, [d16], vals)
        return 0
    lax.fori_loop(0, NJ1, _step, 0)

    pltpu.sync_copy(acc_v, stage_s.at[t])
    plsc.subcore_barrier()

    base = pl.multiple_of(t * ROWS_PER_TILE, 128)
    for r in range(NS):
        pltpu.sync_copy(stage_s.at[r, pl.ds(base, ROWS_PER_TILE)], part_v.at[r])

    def _red(m, _):
        a = part_v[0, pl.ds(m * 16, 16)]
        for r in range(1, NS):
            a = a + part_v[r, pl.ds(m * 16, 16)]
        sum_v[pl.ds(m * 16, 16)] = a
        return 0
    lax.fori_loop(0, ROWS_PER_TILE // 16, _red, 0)

    pltpu.sync_copy(sum_v, out_h.at[c, pl.ds(base, ROWS_PER_TILE)])


_pass2 = pl.kernel(
    _p2_body,
    out_type=jax.ShapeDtypeStruct((NC1, NPAD), jnp.float32),
    mesh=_mesh1,
    scratch_types=[
        pltpu.VMEM((NPAD,), jnp.float32),
        pltpu.VMEM((NPAD,), jnp.float32),
        pltpu.VMEM((NJ1, CH1), jnp.int32),
        pltpu.VMEM((NJ1, CH1), jnp.int32),
        pltpu.VMEM((NS, ROWS_PER_TILE), jnp.float32),
        pltpu.VMEM((ROWS_PER_TILE,), jnp.float32),
        pltpu.VMEM_SHARED((NS, NPAD), jnp.float32),
    ],
    compiler_params=pltpu.CompilerParams(needs_layout_passes=False),
)


def _fin_body(part, pre, deg, o_ref):
    ps = part[0] + part[1]
    z = pre[...] + ps / jnp.maximum(deg[...], 1.0)
    o_ref[...] = 1.0 / (1.0 + jnp.exp(-z))


_finalize = pl.pallas_call(
    _fin_body,
    out_shape=jax.ShapeDtypeStruct((NPAD // 128, 128), jnp.float32),
)


def kernel(features, edge_index, edge_types,
           W1_self, b1_self, W1_neigh, b1_neigh,
           W2_self, b2_self, W2_neigh, b2_neigh):
    src = edge_index[0].astype(jnp.int32)
    dst = edge_index[1].astype(jnp.int32)
    srcp = jnp.concatenate(
        [src, jnp.zeros((EPAD - E,), jnp.int32)])
    dstp = jnp.concatenate(
        [dst, jnp.full((EPAD - E,), NPAD - 1, jnp.int32)])
    src1 = srcp.reshape(NC1 * NS, NJ1, CH1)
    dst1 = dstp.reshape(NC1 * NS, NJ1, CH1)
    src1c = srcp.reshape(NS, NJP1, CH1)
    dst1c = dstp.reshape(NS, NJP1, CH1)
    featp = jnp.pad(features, ((0, NPAD - N), (0, 0)))
    featbf = featp.astype(jnp.bfloat16)
    feat2 = featbf.reshape(NPAD, 2, DH).transpose(1, 0, 2)
    zeros = jnp.zeros((NPAD, DH), jnp.bfloat16)

    agg, degp = _pass1(src1, dst1, feat2, zeros)
    pre, pn, deg = _dense(
        featp, agg, degp, W1_self, W1_neigh,
        W2_self.reshape(1, D), W2_neigh.reshape(1, D),
        b1_self.reshape(1, D), b1_neigh.reshape(1, D),
        b2_self.reshape(1, 1), b2_neigh.reshape(1, 1))
    part = _pass2(src1, dst1, pn.reshape(NPAD))
    out = _finalize(part.reshape(NC1, NPAD // 128, 128),
                    pre.reshape(NPAD // 128, 128),
                    deg.reshape(NPAD // 128, 128))
    return out.reshape(NPAD)[:N]


# speedup vs baseline: 1.6883x; 1.0004x over previous
import functools

import jax
import jax.numpy as jnp
from jax import lax
from jax.experimental import pallas as pl
from jax.experimental.pallas import tpu as pltpu
from jax.experimental.pallas import tpu_sc as plsc

N = 10000
D = 128
NPAD = 10240
E = 320000
EPAD = 327680
NC1, NS = 2, 16
ROWS_PER_TILE = NPAD // NS
CH1 = 128
NJ1 = EPAD // (NC1 * NS) // CH1

_mesh1 = plsc.VectorSubcoreMesh(
    core_axis_name="c", subcore_axis_name="s", num_cores=2, num_subcores=16)

DH = D // 2
NBUF = 8


def _p1_body(src_h, dst_h, feat_h, zeros_h, agg_h, degp_h,
             src_v, dst_v, rows_v, deg_v, tab_s, acc_s, sem_g, sem_s):
    c = lax.axis_index("c")
    s = lax.axis_index("s")
    wid = s * NC1 + c
    z16 = jnp.zeros((16,), jnp.float32)
    o16 = jnp.ones((16,), jnp.float32)

    def _zdeg(i, _):
        deg_v[pl.ds(i * 16, 16)] = z16
        return 0
    lax.fori_loop(0, NPAD // 16, _zdeg, 0)

    pltpu.sync_copy(src_h.at[wid], src_v)
    pltpu.sync_copy(dst_h.at[wid], dst_v)

    base = pl.multiple_of(s * ROWS_PER_TILE, 128)

    rows = tuple(rows_v.at[b] for b in range(NBUF))
    gsem = tuple(sem_g.at[b] for b in range(NBUF))
    ssem = tuple(sem_s.at[b] for b in range(NBUF))

    for h in range(2):
        pltpu.sync_copy(feat_h.at[h, pl.ds(base, ROWS_PER_TILE)],
                        tab_s.at[pl.ds(base, ROWS_PER_TILE)])
        pltpu.sync_copy(zeros_h.at[pl.ds(base, ROWS_PER_TILE)],
                        acc_s.at[pl.ds(base, ROWS_PER_TILE)])
        plsc.subcore_barrier()

        def _gather(j, b):
            return pltpu.async_copy(tab_s.at[src_v.at[j]], rows[b], gsem[b])

        def _scatter(j, b):
            return pltpu.async_copy(rows[b], acc_s.at[dst_v.at[j]], ssem[b],
                                    add=True)

        def _hist(j):
            for k in range(8):
                d16 = dst_v[j, pl.ds(k * 16, 16)]
                plsc.addupdate_scatter(deg_v, [d16], o16)

        def _wait_scatter(j, b):
            pltpu.make_async_copy(rows[b], acc_s.at[dst_v.at[j]],
                                  ssem[b]).wait()

        def _wait_gather(j, b):
            pltpu.make_async_copy(tab_s.at[src_v.at[j]], rows[b],
                                  gsem[b]).wait()

        for b in range(NBUF):
            _gather(b, b)
            if h == 0:
                _hist(b)
        for b in range(NBUF):
            _wait_gather(b, b)
            _scatter(b, b)

        def _grp(jj, _):
            j0 = jj * NBUF
            for b in range(NBUF):
                _wait_scatter(j0 + b - NBUF, b)
                _gather(j0 + b, b)
            if h == 0:
                for b in range(NBUF):
                    _hist(j0 + b)
            for b in range(NBUF):
                _wait_gather(j0 + b, b)
                _scatter(j0 + b, b)
            return 0
        lax.fori_loop(1, NJ1 // NBUF, _grp, 0)
        for b in range(NBUF):
            _wait_scatter(NJ1 - NBUF + b, b)

        plsc.subcore_barrier()
        pltpu.sync_copy(acc_s.at[pl.ds(base, ROWS_PER_TILE)],
                        agg_h.at[c, h, pl.ds(base, ROWS_PER_TILE)])
        plsc.subcore_barrier()
    pltpu.sync_copy(deg_v, degp_h.at[wid])


_pass1 = pl.kernel(
    _p1_body,
    out_type=[
        jax.ShapeDtypeStruct((NC1, 2, NPAD, DH), jnp.bfloat16),
        jax.ShapeDtypeStruct((NC1 * NS, NPAD), jnp.float32),
    ],
    mesh=_mesh1,
    scratch_types=[
        pltpu.VMEM((NJ1, CH1), jnp.int32),
        pltpu.VMEM((NJ1, CH1), jnp.int32),
        pltpu.VMEM((NBUF, CH1, DH), jnp.bfloat16),
        pltpu.VMEM((NPAD,), jnp.float32),
        pltpu.VMEM_SHARED((NPAD, DH), jnp.bfloat16),
        pltpu.VMEM_SHARED((NPAD, DH), jnp.bfloat16),
        pltpu.SemaphoreType.DMA((NBUF,)),
        pltpu.SemaphoreType.DMA((NBUF,)),
    ],
    compiler_params=pltpu.CompilerParams(
        needs_layout_passes=False, use_tc_tiling_on_sc=False),
)


def _dense_body(feat, aggp, degp, w1s, w1n, w2s, w2n, b1s, b1n, b2s, b2n,
                pre_o, pn_o, deg_o):
    deg = jnp.sum(degp[...], axis=0)
    degc = jnp.maximum(deg, 1.0)
    agg = jnp.concatenate(
        [aggp[0, 0].astype(jnp.float32) + aggp[1, 0].astype(jnp.float32),
         aggp[0, 1].astype(jnp.float32) + aggp[1, 1].astype(jnp.float32)],
        axis=1)
    mean = agg * (1.0 / degc)[:, None]
    hp = jax.lax.Precision.HIGHEST
    x = (jnp.dot(feat[...], w1s[...], precision=hp)
         + jnp.dot(mean, w1n[...], precision=hp)
         + b1s[...] + b1n[...])
    x = jnp.maximum(x, 0.0)
    pre_o[...] = (jnp.sum(x * w2s[...], axis=1, keepdims=True)
                  + b2s[...] + b2n[...])
    pn_o[...] = jnp.sum(x * w2n[...], axis=1, keepdims=True)
    deg_o[...] = deg[:, None]


_R = 2048


def _dense(featp, agg, degp, W1_self, W1_neigh, w2s, w2n, b1s, b1n, b2s, b2n):
    grid = (NPAD // _R,)
    full = lambda *shape: pl.BlockSpec(shape, lambda i: (0,) * len(shape))
    return pl.pallas_call(
        _dense_body,
        grid=grid,
        in_specs=[
            pl.BlockSpec((_R, D), lambda i: (i, 0)),
            pl.BlockSpec((NC1, 2, _R, DH), lambda i: (0, 0, i, 0)),
            pl.BlockSpec((NC1 * NS, _R), lambda i: (0, i)),
            full(D, D), full(D, D), full(1, D), full(1, D),
            full(1, D), full(1, D), full(1, 1), full(1, 1),
        ],
        out_specs=[
            pl.BlockSpec((_R, 1), lambda i: (i, 0)),
            pl.BlockSpec((_R, 1), lambda i: (i, 0)),
            pl.BlockSpec((_R, 1), lambda i: (i, 0)),
        ],
        out_shape=[
            jax.ShapeDtypeStruct((NPAD, 1), jnp.float32),
            jax.ShapeDtypeStruct((NPAD, 1), jnp.float32),
            jax.ShapeDtypeStruct((NPAD, 1), jnp.float32),
        ],
    )(featp, agg, degp, W1_self, W1_neigh, w2s, w2n, b1s, b1n, b2s, b2n)


def _p2_body(src_h, dst_h, p_h, out_h,
             p_v, acc_v, src_v, dst_v, part_v, sum_v, stage_s):
    c = lax.axis_index("c")
    t = lax.axis_index("s")
    wid = t * NC1 + c
    z16 = jnp.zeros((16,), jnp.float32)

    pltpu.sync_copy(p_h, p_v)

    def _zacc(i, _):
        acc_v[pl.ds(i * 16, 16)] = z16
        return 0
    lax.fori_loop(0, NPAD // 16, _zacc, 0)

    pltpu.sync_copy(src_h.at[wid], src_v)
    pltpu.sync_copy(dst_h.at[wid], dst_v)

    def _step(j, _):
        for k in range(8):
            s16 = src_v[j, pl.ds(k * 16, 16)]
            d16 = dst_v[j, pl.ds(k * 16, 16)]
            vals = plsc.load_gather(p_v, [s16])
            plsc.addupdate_scatter(acc_v, [d16], vals)
        return 0
    lax.fori_loop(0, NJ1, _step, 0)

    pltpu.sync_copy(acc_v, stage_s.at[t])
    plsc.subcore_barrier()

    base = pl.multiple_of(t * ROWS_PER_TILE, 128)
    for r in range(NS):
        pltpu.sync_copy(stage_s.at[r, pl.ds(base, ROWS_PER_TILE)], part_v.at[r])

    def _red(m, _):
        a = part_v[0, pl.ds(m * 16, 16)]
        for r in range(1, NS):
            a = a + part_v[r, pl.ds(m * 16, 16)]
        sum_v[pl.ds(m * 16, 16)] = a
        return 0
    lax.fori_loop(0, ROWS_PER_TILE // 16, _red, 0)

    pltpu.sync_copy(sum_v, out_h.at[c, pl.ds(base, ROWS_PER_TILE)])


_pass2 = pl.kernel(
    _p2_body,
    out_type=jax.ShapeDtypeStruct((NC1, NPAD), jnp.float32),
    mesh=_mesh1,
    scratch_types=[
        pltpu.VMEM((NPAD,), jnp.float32),
        pltpu.VMEM((NPAD,), jnp.float32),
        pltpu.VMEM((NJ1, CH1), jnp.int32),
        pltpu.VMEM((NJ1, CH1), jnp.int32),
        pltpu.VMEM((NS, ROWS_PER_TILE), jnp.float32),
        pltpu.VMEM((ROWS_PER_TILE,), jnp.float32),
        pltpu.VMEM_SHARED((NS, NPAD), jnp.float32),
    ],
    compiler_params=pltpu.CompilerParams(needs_layout_passes=False),
)


def _fin_body(part, pre, deg, o_ref):
    ps = part[0] + part[1]
    z = pre[...] + ps / jnp.maximum(deg[...], 1.0)
    o_ref[...] = 1.0 / (1.0 + jnp.exp(-z))


_finalize = pl.pallas_call(
    _fin_body,
    out_shape=jax.ShapeDtypeStruct((NPAD // 128, 128), jnp.float32),
)


def kernel(features, edge_index, edge_types,
           W1_self, b1_self, W1_neigh, b1_neigh,
           W2_self, b2_self, W2_neigh, b2_neigh):
    src = edge_index[0].astype(jnp.int32)
    dst = edge_index[1].astype(jnp.int32)
    srcp = jnp.concatenate(
        [src, jnp.zeros((EPAD - E,), jnp.int32)])
    dstp = jnp.concatenate(
        [dst, jnp.full((EPAD - E,), NPAD - 1, jnp.int32)])
    src1 = srcp.reshape(NC1 * NS, NJ1, CH1)
    dst1 = dstp.reshape(NC1 * NS, NJ1, CH1)
    featp = jnp.pad(features, ((0, NPAD - N), (0, 0)))
    featbf = featp.astype(jnp.bfloat16)
    feat2 = featbf.reshape(NPAD, 2, DH).transpose(1, 0, 2)
    zeros = jnp.zeros((NPAD, DH), jnp.bfloat16)

    agg, degp = _pass1(src1, dst1, feat2, zeros)
    pre, pn, deg = _dense(
        featp, agg, degp, W1_self, W1_neigh,
        W2_self.reshape(1, D), W2_neigh.reshape(1, D),
        b1_self.reshape(1, D), b1_neigh.reshape(1, D),
        b2_self.reshape(1, 1), b2_neigh.reshape(1, 1))
    part = _pass2(src1, dst1, pn.reshape(NPAD))
    out = _finalize(part.reshape(NC1, NPAD // 128, 128),
                    pre.reshape(NPAD // 128, 128),
                    deg.reshape(NPAD // 128, 128))
    return out.reshape(NPAD)[:N]
